# baseline (speedup 1.0000x reference)
import functools

import jax
import jax.numpy as jnp
from jax import lax
from jax.experimental import pallas as pl
from jax.experimental.pallas import tpu as pltpu
from jax.experimental.pallas import tpu_sc as plsc

_K = 32
_NBINS = 256


def _encode_body(x_ref, w_ref, b_ref, a_ref):
    z = jax.lax.dot_general(
        x_ref[...], w_ref[...], (((1,), (1,)), ((), ())),
        preferred_element_type=jnp.float32)
    z = z + b_ref[...]
    a_ref[...] = jnp.where(z > 0.0, z, 0.0)


def _select_decode_body(a_ref, thr_ref, wdt_ref, bd_ref, sp_ref, xh_ref):
    a = a_ref[...]
    t = thr_ref[:, 0:1]
    s = jnp.where(a >= t, a, 0.0)
    sp_ref[...] = s
    xh = jax.lax.dot_general(
        s.astype(jnp.bfloat16), wdt_ref[...], (((1,), (0,)), ((), ())),
        preferred_element_type=jnp.float32)
    xh_ref[...] = xh + bd_ref[...]


def _sc_thresholds(a, n, d_dict):
    info = plsc.get_sparse_core_info()
    nw = info.num_cores * info.num_subcores
    lanes = info.num_lanes
    rpw = n // nw
    nvec = d_dict // lanes
    mesh = plsc.VectorSubcoreMesh(core_axis_name="c", subcore_axis_name="s")

    @functools.partial(
        pl.kernel, mesh=mesh,
        out_type=jax.ShapeDtypeStruct((n,), jnp.int32),
        compiler_params=pltpu.CompilerParams(needs_layout_passes=False),
        scratch_types=[
            pltpu.VMEM((d_dict,), jnp.int32),
            pltpu.VMEM((_NBINS * lanes,), jnp.int32),
            pltpu.VMEM((_NBINS * lanes,), jnp.int32),
            pltpu.VMEM((rpw,), jnp.int32),
        ],
    )
    def k(a_hbm, thr_hbm, row_v, bins_v, sfx_v, tout_v):
        wid = lax.axis_index("s") * info.num_cores + lax.axis_index("c")
        base = wid * rpw
        lane_iota = lax.iota(jnp.int32, lanes)
        ones = jnp.ones((lanes,), jnp.int32)
        zeros16 = jnp.zeros((lanes,), jnp.int32)

        def zero_bins(j, carry):
            bins_v[pl.ds(j * lanes, lanes)] = zeros16
            return carry

        lax.fori_loop(0, _NBINS, zero_bins, 0)

        def row_step(r, carry):
            pltpu.sync_copy(a_hbm.at[base + r], row_v)

            def digit_pass(shift, width, prefix, kk):
                pshift = shift + width
                dmask = (1 << width) - 1

                def scatter(j, c):
                    vi = row_v[pl.ds(j * lanes, lanes)]
                    digit = (vi >> shift) & dmask
                    idx = digit * lanes + lane_iota
                    keep = (vi >> pshift) == prefix
                    plsc.addupdate_scatter(bins_v, [idx], ones, mask=keep)
                    return c

                lax.fori_loop(0, nvec, scatter, 0)

                def sweep(jj, acc):
                    b = (_NBINS - 1) - jj
                    h = bins_v[pl.ds(b * lanes, lanes)]
                    acc = acc + h
                    bins_v[pl.ds(b * lanes, lanes)] = zeros16
                    sfx_v[pl.ds(b * lanes, lanes)] = acc
                    return acc

                lax.fori_loop(0, _NBINS, sweep, zeros16)

                def cnt_ge(b):
                    return jnp.sum(sfx_v[pl.ds(b * lanes, lanes)])

                def bsearch(stp, lo):
                    mid = lo + (128 >> stp)
                    safe = jnp.minimum(mid, _NBINS - 1)
                    ok = (mid <= _NBINS - 1) & (cnt_ge(safe) >= kk)
                    return jnp.where(ok, mid, lo)

                bbin = lax.fori_loop(0, 8, bsearch, jnp.int32(0))
                above = jnp.where(bbin >= _NBINS - 1, 0,
                                  cnt_ge(jnp.minimum(bbin + 1, _NBINS - 1)))
                return (prefix << width) | bbin, kk - above

            prefix = jnp.int32(0)
            kk = jnp.int32(_K)
            for shift, width in ((23, 8), (15, 8), (7, 8), (0, 7)):
                prefix, kk = digit_pass(shift, width, prefix, kk)
            plsc.store_scatter(tout_v, [jnp.broadcast_to(r, (lanes,))],
                               jnp.broadcast_to(prefix, (lanes,)),
                               mask=lane_iota == 0)
            return carry

        lax.fori_loop(0, rpw, row_step, 0)
        pltpu.sync_copy(tout_v, thr_hbm.at[pl.ds(base, rpw)])

    return k(a)


def kernel(x, W_enc, b_enc, W_dec, b_dec):
    n, d_model = x.shape
    d_dict = W_enc.shape[0]
    bre = min(512, n)
    bc = min(2048, d_dict)
    br2 = min(128, n)

    a = pl.pallas_call(
        _encode_body,
        grid=(d_dict // bc, n // bre),
        in_specs=[
            pl.BlockSpec((bre, d_model), lambda cb, rb: (rb, 0)),
            pl.BlockSpec((bc, d_model), lambda cb, rb: (cb, 0)),
            pl.BlockSpec((1, bc), lambda cb, rb: (0, cb)),
        ],
        out_specs=pl.BlockSpec((bre, bc), lambda cb, rb: (rb, cb)),
        out_shape=jax.ShapeDtypeStruct((n, d_dict), jnp.float32),
    )(x, W_enc, b_enc.reshape(1, d_dict))

    thr_bits = _sc_thresholds(lax.bitcast_convert_type(a, jnp.int32),
                              n, d_dict)
    thr = lax.bitcast_convert_type(thr_bits, jnp.float32)
    thr_b = jnp.broadcast_to(thr[:, None], (n, 128))

    wdt = W_dec.T.astype(jnp.bfloat16)
    sparse, x_hat = pl.pallas_call(
        _select_decode_body,
        grid=(n // br2,),
        in_specs=[
            pl.BlockSpec((br2, d_dict), lambda i: (i, 0)),
            pl.BlockSpec((br2, 128), lambda i: (i, 0)),
            pl.BlockSpec((d_dict, d_model), lambda i: (0, 0)),
            pl.BlockSpec((1, d_model), lambda i: (0, 0)),
        ],
        out_specs=[
            pl.BlockSpec((br2, d_dict), lambda i: (i, 0)),
            pl.BlockSpec((br2, d_model), lambda i: (i, 0)),
        ],
        out_shape=[
            jax.ShapeDtypeStruct((n, d_dict), jnp.float32),
            jax.ShapeDtypeStruct((n, d_model), jnp.float32),
        ],
    )(a, thr_b, wdt, b_dec.reshape(1, d_model))
    return (x_hat, sparse)


# speedup vs baseline: 3.6460x; 3.6460x over previous
import functools

import jax
import jax.numpy as jnp
from jax import lax
from jax.experimental import pallas as pl
from jax.experimental.pallas import tpu as pltpu
from jax.experimental.pallas import tpu_sc as plsc

_K = 32
_NBINS = 256
_SC_ROWS = 512


def _encode_body(x_ref, w_ref, b_ref, a_ref):
    z = jax.lax.dot_general(
        x_ref[...], w_ref[...], (((1,), (1,)), ((), ())),
        preferred_element_type=jnp.float32)
    z = z + b_ref[...]
    a_ref[...] = jnp.where(z > 0.0, z, 0.0)


def _bisect_decode_body(a_ref, wdt_ref, bd_ref, sp_ref, xh_ref):
    a = a_ref[...]
    ai = jax.lax.bitcast_convert_type(a, jnp.int32)
    rows = a.shape[0]

    def bit_step(b, t):
        cand = t | jax.lax.shift_left(1, 30 - b)
        cnt = jnp.sum((ai >= cand).astype(jnp.int32), axis=1, keepdims=True)
        return jnp.where(cnt >= _K, cand, t)

    t = jax.lax.fori_loop(0, 31, bit_step, jnp.zeros((rows, 1), jnp.int32))
    s = jnp.where(ai >= t, a, 0.0)
    sp_ref[...] = s
    xh = jax.lax.dot_general(
        s.astype(jnp.bfloat16), wdt_ref[...], (((1,), (0,)), ((), ())),
        preferred_element_type=jnp.float32)
    xh_ref[...] = xh + bd_ref[...]


def _mask_decode_body(a_ref, thr_ref, wdt_ref, bd_ref, sp_ref, xh_ref):
    a = a_ref[...]
    t = thr_ref[:, 0:1]
    s = jnp.where(a >= t, a, 0.0)
    sp_ref[...] = s
    xh = jax.lax.dot_general(
        s.astype(jnp.bfloat16), wdt_ref[...], (((1,), (0,)), ((), ())),
        preferred_element_type=jnp.float32)
    xh_ref[...] = xh + bd_ref[...]


def _sc_thresholds(a_bits, n, d_dict):
    info = plsc.get_sparse_core_info()
    nw = info.num_cores * info.num_subcores
    lanes = info.num_lanes
    rpw = n // nw
    nvec = d_dict // lanes
    mesh = plsc.VectorSubcoreMesh(core_axis_name="c", subcore_axis_name="s")

    @functools.partial(
        pl.kernel, mesh=mesh,
        out_type=jax.ShapeDtypeStruct((n,), jnp.int32),
        compiler_params=pltpu.CompilerParams(needs_layout_passes=False),
        scratch_types=[
            pltpu.VMEM((d_dict,), jnp.int32),
            pltpu.VMEM((_NBINS * lanes,), jnp.int32),
            pltpu.VMEM((_NBINS * lanes,), jnp.int32),
            pltpu.VMEM((rpw,), jnp.int32),
        ],
    )
    def k(a_hbm, thr_hbm, row_v, bins_v, sfx_v, tout_v):
        wid = lax.axis_index("s") * info.num_cores + lax.axis_index("c")
        base = wid * rpw
        lane_iota = lax.iota(jnp.int32, lanes)
        ones = jnp.ones((lanes,), jnp.int32)
        zeros16 = jnp.zeros((lanes,), jnp.int32)

        def zero_bins(j, carry):
            bins_v[pl.ds(j * lanes, lanes)] = zeros16
            return carry

        lax.fori_loop(0, _NBINS, zero_bins, 0)

        def row_step(r, carry):
            pltpu.sync_copy(a_hbm.at[base + r], row_v)

            def digit_pass(shift, width, prefix, kk):
                pshift = shift + width
                dmask = (1 << width) - 1

                def scatter(j, c):
                    vi = row_v[pl.ds(j * lanes, lanes)]
                    digit = (vi >> shift) & dmask
                    idx = digit * lanes + lane_iota
                    keep = (vi >> pshift) == prefix
                    plsc.addupdate_scatter(bins_v, [idx], ones, mask=keep)
                    return c

                lax.fori_loop(0, nvec, scatter, 0)

                def sweep(jj, acc):
                    b = (_NBINS - 1) - jj
                    h = bins_v[pl.ds(b * lanes, lanes)]
                    acc = acc + h
                    bins_v[pl.ds(b * lanes, lanes)] = zeros16
                    sfx_v[pl.ds(b * lanes, lanes)] = acc
                    return acc

                lax.fori_loop(0, _NBINS, sweep, zeros16)

                def cnt_ge(b):
                    return jnp.sum(sfx_v[pl.ds(b * lanes, lanes)])

                def bsearch(stp, lo):
                    mid = lo + (128 >> stp)
                    safe = jnp.minimum(mid, _NBINS - 1)
                    ok = (mid <= _NBINS - 1) & (cnt_ge(safe) >= kk)
                    return jnp.where(ok, mid, lo)

                bbin = lax.fori_loop(0, 8, bsearch, jnp.int32(0))
                above = jnp.where(bbin >= _NBINS - 1, 0,
                                  cnt_ge(jnp.minimum(bbin + 1, _NBINS - 1)))
                return (prefix << width) | bbin, kk - above

            prefix = jnp.int32(0)
            kk = jnp.int32(_K)
            for shift, width in ((23, 8), (15, 8), (7, 8), (0, 7)):
                prefix, kk = digit_pass(shift, width, prefix, kk)
            plsc.store_scatter(tout_v, [jnp.broadcast_to(r, (lanes,))],
                               jnp.broadcast_to(prefix, (lanes,)),
                               mask=lane_iota == 0)
            return carry

        lax.fori_loop(0, rpw, row_step, 0)
        pltpu.sync_copy(tout_v, thr_hbm.at[pl.ds(base, rpw)])

    return k(a_bits)


def kernel(x, W_enc, b_enc, W_dec, b_dec):
    n, d_model = x.shape
    d_dict = W_enc.shape[0]
    bre = min(512, n)
    bc = min(2048, d_dict)
    br2 = min(128, n)
    nsc = _SC_ROWS if n % _SC_ROWS == 0 and n > _SC_ROWS else 0

    a = pl.pallas_call(
        _encode_body,
        grid=(d_dict // bc, n // bre),
        in_specs=[
            pl.BlockSpec((bre, d_model), lambda cb, rb: (rb, 0)),
            pl.BlockSpec((bc, d_model), lambda cb, rb: (cb, 0)),
            pl.BlockSpec((1, bc), lambda cb, rb: (0, cb)),
        ],
        out_specs=pl.BlockSpec((bre, bc), lambda cb, rb: (rb, cb)),
        out_shape=jax.ShapeDtypeStruct((n, d_dict), jnp.float32),
    )(x, W_enc, b_enc.reshape(1, d_dict))

    wdt = W_dec.T.astype(jnp.bfloat16)
    bd = b_dec.reshape(1, d_model)

    a_tc = lax.slice(a, (nsc, 0), (n, d_dict))
    n_tc = n - nsc
    sp_tc, xh_tc = pl.pallas_call(
        _bisect_decode_body,
        grid=(n_tc // br2,),
        in_specs=[
            pl.BlockSpec((br2, d_dict), lambda i: (i, 0)),
            pl.BlockSpec((d_dict, d_model), lambda i: (0, 0)),
            pl.BlockSpec((1, d_model), lambda i: (0, 0)),
        ],
        out_specs=[
            pl.BlockSpec((br2, d_dict), lambda i: (i, 0)),
            pl.BlockSpec((br2, d_model), lambda i: (i, 0)),
        ],
        out_shape=[
            jax.ShapeDtypeStruct((n_tc, d_dict), jnp.float32),
            jax.ShapeDtypeStruct((n_tc, d_model), jnp.float32),
        ],
    )(a_tc, wdt, bd)

    if nsc:
        a_sc = lax.slice(a, (0, 0), (nsc, d_dict))
        thr_bits = _sc_thresholds(
            lax.bitcast_convert_type(a_sc, jnp.int32), nsc, d_dict)
        thr = lax.bitcast_convert_type(thr_bits, jnp.float32)
        thr_b = jnp.broadcast_to(thr[:, None], (nsc, 128))
        sp_sc, xh_sc = pl.pallas_call(
            _mask_decode_body,
            grid=(nsc // br2,),
            in_specs=[
                pl.BlockSpec((br2, d_dict), lambda i: (i, 0)),
                pl.BlockSpec((br2, 128), lambda i: (i, 0)),
                pl.BlockSpec((d_dict, d_model), lambda i: (0, 0)),
                pl.BlockSpec((1, d_model), lambda i: (0, 0)),
            ],
            out_specs=[
                pl.BlockSpec((br2, d_dict), lambda i: (i, 0)),
                pl.BlockSpec((br2, d_model), lambda i: (i, 0)),
            ],
            out_shape=[
                jax.ShapeDtypeStruct((nsc, d_dict), jnp.float32),
                jax.ShapeDtypeStruct((nsc, d_model), jnp.float32),
            ],
        )(a_sc, thr_b, wdt, bd)
        sparse = lax.concatenate((sp_sc, sp_tc), 0)
        x_hat = lax.concatenate((xh_sc, xh_tc), 0)
    else:
        sparse, x_hat = sp_tc, xh_tc
    return (x_hat, sparse)


# speedup vs baseline: 4.8046x; 1.3178x over previous
import jax
import jax.numpy as jnp
from jax.experimental import pallas as pl

_K = 32


def _encode_body(x_ref, w_ref, b_ref, a_ref):
    z = jax.lax.dot_general(
        x_ref[...], w_ref[...], (((1,), (1,)), ((), ())),
        preferred_element_type=jnp.float32)
    z = z + b_ref[...]
    a_ref[...] = jnp.where(z > 0.0, z, 0.0)


def _select_decode_body(a_ref, wdt_ref, bd_ref, sp_ref, xh_ref):
    a = a_ref[...]
    ai = jax.lax.bitcast_convert_type(a, jnp.int32)
    rows = a.shape[0]

    def bit_step(b, t):
        cand = t | jax.lax.shift_left(1, 30 - b)
        cnt = jnp.sum((ai >= cand).astype(jnp.int32), axis=1, keepdims=True)
        return jnp.where(cnt >= _K, cand, t)

    t = jax.lax.fori_loop(0, 31, bit_step, jnp.zeros((rows, 1), jnp.int32))
    s = jnp.where(ai >= t, a, 0.0)
    sp_ref[...] = s
    xh = jax.lax.dot_general(
        s.astype(jnp.bfloat16), wdt_ref[...], (((1,), (0,)), ((), ())),
        preferred_element_type=jnp.float32)
    xh_ref[...] = xh + bd_ref[...]


def kernel(x, W_enc, b_enc, W_dec, b_dec):
    n, d_model = x.shape
    d_dict = W_enc.shape[0]
    bre = min(512, n)
    bc = min(2048, d_dict)
    br2 = min(256, n)

    a = pl.pallas_call(
        _encode_body,
        grid=(d_dict // bc, n // bre),
        in_specs=[
            pl.BlockSpec((bre, d_model), lambda cb, rb: (rb, 0)),
            pl.BlockSpec((bc, d_model), lambda cb, rb: (cb, 0)),
            pl.BlockSpec((1, bc), lambda cb, rb: (0, cb)),
        ],
        out_specs=pl.BlockSpec((bre, bc), lambda cb, rb: (rb, cb)),
        out_shape=jax.ShapeDtypeStruct((n, d_dict), jnp.float32),
    )(x, W_enc, b_enc.reshape(1, d_dict))

    wdt = W_dec.T.astype(jnp.bfloat16)
    sparse, x_hat = pl.pallas_call(
        _select_decode_body,
        grid=(n // br2,),
        in_specs=[
            pl.BlockSpec((br2, d_dict), lambda i: (i, 0)),
            pl.BlockSpec((d_dict, d_model), lambda i: (0, 0)),
            pl.BlockSpec((1, d_model), lambda i: (0, 0)),
        ],
        out_specs=[
            pl.BlockSpec((br2, d_dict), lambda i: (i, 0)),
            pl.BlockSpec((br2, d_model), lambda i: (i, 0)),
        ],
        out_shape=[
            jax.ShapeDtypeStruct((n, d_dict), jnp.float32),
            jax.ShapeDtypeStruct((n, d_model), jnp.float32),
        ],
    )(a, wdt, b_dec.reshape(1, d_model))
    return (x_hat, sparse)
